# Initial kernel scaffold; baseline (speedup 1.0000x reference)
import jax
import jax.numpy as jnp
from jax.experimental import pallas as pl


def kernel(x, means):
    raise NotImplementedError("write your pallas kernel here")


# speedup vs baseline: 15.6515x; 15.6515x over previous
import functools

import jax
import jax.numpy as jnp
from jax.experimental import pallas as pl
from jax.experimental.pallas import tpu as pltpu

_EPS = 1e-6
_COMMITMENT = 1e-4


def _dist_block_kernel(x_ref, means_ref, dists_ref, dmin_ref):
    x = x_ref[0]
    m = means_ref[0]
    nrm = jnp.sqrt(jnp.sum(x * x, axis=1, keepdims=True))
    xn = x / (nrm + _EPS)
    x2 = jnp.sum(xn * xn, axis=1, keepdims=True)
    m2 = jnp.sum(m * m, axis=1)[None, :]
    xm = jax.lax.dot_general(xn, m, (((1,), (1,)), ((), ())),
                             preferred_element_type=jnp.float32)
    d2 = (x2 + m2) - 2.0 * xm
    dists_ref[0] = jnp.sqrt(jnp.maximum(d2, 0.0))
    dmin_ref[0, 0] = jnp.min(d2, axis=1)[None, :]


def _loss_reduce_kernel(dmin_ref, out_ref, *, d):
    n_tokens = dmin_ref.shape[0] * dmin_ref.shape[1]
    scale = _COMMITMENT / float(n_tokens * d)
    s = jnp.sum(dmin_ref[...])
    out_ref[...] = jnp.broadcast_to(s * scale, out_ref.shape)


def kernel(x, means):
    H, T, D = x.shape
    C = means.shape[1]
    Tt = 1024
    nT = T // Tt

    dists, dmin = pl.pallas_call(
        _dist_block_kernel,
        grid=(H, nT),
        in_specs=[
            pl.BlockSpec((1, Tt, D), lambda h, t: (h, t, 0)),
            pl.BlockSpec((1, C, D), lambda h, t: (h, 0, 0)),
        ],
        out_specs=[
            pl.BlockSpec((1, Tt, C), lambda h, t: (h, t, 0)),
            pl.BlockSpec((1, 1, 1, Tt), lambda h, t: (h, t, 0, 0)),
        ],
        out_shape=[
            jax.ShapeDtypeStruct((H, T, C), jnp.float32),
            jax.ShapeDtypeStruct((H, nT, 1, Tt), jnp.float32),
        ],
        compiler_params=pltpu.CompilerParams(
            dimension_semantics=("parallel", "parallel")),
    )(x, means)

    loss_tile = pl.pallas_call(
        functools.partial(_loss_reduce_kernel, d=D),
        out_shape=jax.ShapeDtypeStruct((8, 128), jnp.float32),
    )(dmin.reshape(H * nT, Tt))
    return dists, loss_tile[0, 0]
